# baseline (speedup 1.0000x reference)
import jax
import jax.numpy as jnp
from jax import lax
import jax.experimental.pallas as pl
from jax.experimental.pallas import tpu as pltpu
from jax.experimental.pallas import tpu_sc as plsc

H = 128
TBATCH_BLK = 128

_SC_INFO = plsc.get_sparse_core_info()
NC = _SC_INFO.num_cores
NS = _SC_INFO.num_subcores
NW = NC * NS

NBUF = 4


def _tc_img_kernel(img_ref, tbl_ref, img_out_ref):
    img_out_ref[...] = img_ref[...] + tbl_ref[1:2, :][None]


def _tc_img_only(img_embedding, mod_embed_table):
    B, S_img, h = img_embedding.shape
    img_t = jnp.transpose(img_embedding, (1, 0, 2))
    out_t = pl.pallas_call(
        _tc_img_kernel,
        grid=(B // TBATCH_BLK,),
        in_specs=[
            pl.BlockSpec((S_img, TBATCH_BLK, h), lambda i: (0, i, 0)),
            pl.BlockSpec((2, h), lambda i: (0, 0)),
        ],
        out_specs=pl.BlockSpec((S_img, TBATCH_BLK, h), lambda i: (0, i, 0)),
        out_shape=jax.ShapeDtypeStruct((S_img, B, h), img_embedding.dtype),
    )(img_t, mod_embed_table)
    return jnp.transpose(out_t, (1, 0, 2))


def _sc_poi_body(poi_hbm, tbl_hbm, poi_out, tbl_v, bufs, sin, sout):
    wid = lax.axis_index("s") * NC + lax.axis_index("c")
    B, S, _ = poi_hbm.shape
    n = B // NW
    base = wid * n

    pltpu.sync_copy(tbl_hbm, tbl_v)
    row0 = [tbl_v[0, pl.ds(h * 16, 16)] for h in range(H // 16)]

    def start_in(c):
        pltpu.async_copy(poi_hbm.at[pl.ds(base + c, 1)], bufs[c % NBUF], sin[c % NBUF])

    def wait_in(c):
        pltpu.make_async_copy(
            poi_hbm.at[pl.ds(base + c, 1)], bufs[c % NBUF], sin[c % NBUF]
        ).wait()

    def start_out(c):
        pltpu.async_copy(bufs[c % NBUF], poi_out.at[pl.ds(base + c, 1)], sout[c % NBUF])

    def wait_out(c):
        pltpu.make_async_copy(
            bufs[c % NBUF], poi_out.at[pl.ds(base + c, 1)], sout[c % NBUF]
        ).wait()

    def compute(c):
        buf = bufs[c % NBUF]

        @plsc.parallel_loop(0, S, unroll=8)
        def _row(s):
            for h in range(H // 16):
                sl = pl.ds(h * 16, 16)
                buf[0, s, sl] = buf[0, s, sl] + row0[h]

    for k in range(min(NBUF - 1, n)):
        start_in(k)
    for c in range(n):
        wait_in(c)
        nxt = c + NBUF - 1
        if nxt < n:
            if c >= 1:
                wait_out(c - 1)
            start_in(nxt)
        compute(c)
        start_out(c)
    for c in range(max(0, n - NBUF + 1), n):
        wait_out(c)


def _sc_poi_only(poi_embedding, mod_embed_table):
    B, S_poi, h = poi_embedding.shape
    mesh = plsc.VectorSubcoreMesh(core_axis_name="c", subcore_axis_name="s")
    run = pl.kernel(
        _sc_poi_body,
        out_type=jax.ShapeDtypeStruct((B, S_poi, h), jnp.float32),
        mesh=mesh,
        scratch_types=[
            pltpu.VMEM((2, h), jnp.float32),
            [pltpu.VMEM((1, S_poi, h), jnp.float32) for _ in range(NBUF)],
            [pltpu.SemaphoreType.DMA for _ in range(NBUF)],
            [pltpu.SemaphoreType.DMA for _ in range(NBUF)],
        ],
    )
    return run(poi_embedding, mod_embed_table)


def kernel(poi_embedding, img_embedding, mod_embed_table):
    poi_out = _sc_poi_only(poi_embedding, mod_embed_table)
    img_out = _tc_img_only(img_embedding, mod_embed_table)
    return poi_out, img_out


# speedup vs baseline: 1.0930x; 1.0930x over previous
import jax
import jax.numpy as jnp
from jax import lax
import jax.experimental.pallas as pl
from jax.experimental.pallas import tpu as pltpu
from jax.experimental.pallas import tpu_sc as plsc

H = 128
BATCH_BLK = 64

_SC_INFO = plsc.get_sparse_core_info()
NC = _SC_INFO.num_cores
NS = _SC_INFO.num_subcores
NW = NC * NS

NBUF = 4
SC_NB = 32


def _tc_poi_kernel(poi_ref, tbl_ref, poi_out_ref):
    poi_out_ref[...] = poi_ref[...] + tbl_ref[0:1, :][None]


def _tc_poi_only(poi_embedding, mod_embed_table):
    B, S_poi, h = poi_embedding.shape
    return pl.pallas_call(
        _tc_poi_kernel,
        grid=(B // BATCH_BLK,),
        in_specs=[
            pl.BlockSpec((BATCH_BLK, S_poi, h), lambda i: (i, 0, 0)),
            pl.BlockSpec((2, h), lambda i: (0, 0)),
        ],
        out_specs=pl.BlockSpec((BATCH_BLK, S_poi, h), lambda i: (i, 0, 0)),
        out_shape=jax.ShapeDtypeStruct(poi_embedding.shape, poi_embedding.dtype),
    )(poi_embedding, mod_embed_table)


def _sc_img_body(img_hbm, tbl_hbm, img_out, tbl_v, bufs, sin, sout):
    wid = lax.axis_index("s") * NC + lax.axis_index("c")
    S, B, _ = img_hbm.shape
    base = wid * SC_NB
    n = S

    pltpu.sync_copy(tbl_hbm, tbl_v)
    row1 = [tbl_v[1, pl.ds(h * 16, 16)] for h in range(H // 16)]

    def start_in(c):
        pltpu.async_copy(
            img_hbm.at[pl.ds(c, 1), pl.ds(base, SC_NB)], bufs[c % NBUF], sin[c % NBUF]
        )

    def wait_in(c):
        pltpu.make_async_copy(
            img_hbm.at[pl.ds(c, 1), pl.ds(base, SC_NB)], bufs[c % NBUF], sin[c % NBUF]
        ).wait()

    def start_out(c):
        pltpu.async_copy(
            bufs[c % NBUF], img_out.at[pl.ds(c, 1), pl.ds(base, SC_NB)], sout[c % NBUF]
        )

    def wait_out(c):
        pltpu.make_async_copy(
            bufs[c % NBUF], img_out.at[pl.ds(c, 1), pl.ds(base, SC_NB)], sout[c % NBUF]
        ).wait()

    def compute(c):
        buf = bufs[c % NBUF]

        @plsc.parallel_loop(0, SC_NB, unroll=4)
        def _batch(b):
            for h in range(H // 16):
                sl = pl.ds(h * 16, 16)
                buf[0, b, sl] = buf[0, b, sl] + row1[h]

    for k in range(min(NBUF - 1, n)):
        start_in(k)
    for c in range(n):
        wait_in(c)
        nxt = c + NBUF - 1
        if nxt < n:
            if c >= 1:
                wait_out(c - 1)
            start_in(nxt)
        compute(c)
        start_out(c)
    for c in range(max(0, n - NBUF + 1), n):
        wait_out(c)


def _sc_img_only(img_embedding, mod_embed_table):
    B, S_img, h = img_embedding.shape
    img_t = jnp.transpose(img_embedding, (1, 0, 2))
    mesh = plsc.VectorSubcoreMesh(core_axis_name="c", subcore_axis_name="s")
    run = pl.kernel(
        _sc_img_body,
        out_type=jax.ShapeDtypeStruct((S_img, B, h), jnp.float32),
        mesh=mesh,
        scratch_types=[
            pltpu.VMEM((2, h), jnp.float32),
            [pltpu.VMEM((1, SC_NB, h), jnp.float32) for _ in range(NBUF)],
            [pltpu.SemaphoreType.DMA for _ in range(NBUF)],
            [pltpu.SemaphoreType.DMA for _ in range(NBUF)],
        ],
    )
    out_t = run(img_t, mod_embed_table)
    return jnp.transpose(out_t, (1, 0, 2))


def kernel(poi_embedding, img_embedding, mod_embed_table):
    img_out = _sc_img_only(img_embedding, mod_embed_table)
    poi_out = _tc_poi_only(poi_embedding, mod_embed_table)
    return poi_out, img_out


# speedup vs baseline: 1.1102x; 1.0158x over previous
import jax
import jax.numpy as jnp
from jax import lax
import jax.experimental.pallas as pl
from jax.experimental.pallas import tpu as pltpu
from jax.experimental.pallas import tpu_sc as plsc

H = 128
BATCH_BLK = 128

_SC_INFO = plsc.get_sparse_core_info()
NC = _SC_INFO.num_cores
NS = _SC_INFO.num_subcores
NW = NC * NS

NBUF = 4
SC_NB = 32


def _tc_poi_kernel(poi_ref, tbl_ref, poi_out_ref):
    poi_out_ref[...] = poi_ref[...] + tbl_ref[0:1, :][None]


def _tc_poi_only(poi_embedding, mod_embed_table):
    B, S_poi, h = poi_embedding.shape
    return pl.pallas_call(
        _tc_poi_kernel,
        grid=(B // BATCH_BLK,),
        in_specs=[
            pl.BlockSpec((BATCH_BLK, S_poi, h), lambda i: (i, 0, 0)),
            pl.BlockSpec((2, h), lambda i: (0, 0)),
        ],
        out_specs=pl.BlockSpec((BATCH_BLK, S_poi, h), lambda i: (i, 0, 0)),
        out_shape=jax.ShapeDtypeStruct(poi_embedding.shape, poi_embedding.dtype),
    )(poi_embedding, mod_embed_table)


def _sc_img_body(img_hbm, tbl_hbm, img_out, tbl_v, bufs, sin, sout):
    wid = lax.axis_index("s") * NC + lax.axis_index("c")
    S, B, _ = img_hbm.shape
    base = wid * SC_NB
    n = S

    pltpu.sync_copy(tbl_hbm, tbl_v)
    row1 = [tbl_v[1, pl.ds(h * 16, 16)] for h in range(H // 16)]

    def start_in(c):
        pltpu.async_copy(
            img_hbm.at[pl.ds(c, 1), pl.ds(base, SC_NB)], bufs[c % NBUF], sin[c % NBUF]
        )

    def wait_in(c):
        pltpu.make_async_copy(
            img_hbm.at[pl.ds(c, 1), pl.ds(base, SC_NB)], bufs[c % NBUF], sin[c % NBUF]
        ).wait()

    def start_out(c):
        pltpu.async_copy(
            bufs[c % NBUF], img_out.at[pl.ds(c, 1), pl.ds(base, SC_NB)], sout[c % NBUF]
        )

    def wait_out(c):
        pltpu.make_async_copy(
            bufs[c % NBUF], img_out.at[pl.ds(c, 1), pl.ds(base, SC_NB)], sout[c % NBUF]
        ).wait()

    def compute(c):
        buf = bufs[c % NBUF]

        @plsc.parallel_loop(0, SC_NB, unroll=4)
        def _batch(b):
            for h in range(H // 16):
                sl = pl.ds(h * 16, 16)
                buf[0, b, sl] = buf[0, b, sl] + row1[h]

    for k in range(min(NBUF - 1, n)):
        start_in(k)
    for c in range(n):
        wait_in(c)
        nxt = c + NBUF - 1
        if nxt < n:
            if c >= 1:
                wait_out(c - 1)
            start_in(nxt)
        compute(c)
        start_out(c)
    for c in range(max(0, n - NBUF + 1), n):
        wait_out(c)


def _sc_img_only(img_embedding, mod_embed_table):
    B, S_img, h = img_embedding.shape
    img_t = jnp.transpose(img_embedding, (1, 0, 2))
    mesh = plsc.VectorSubcoreMesh(core_axis_name="c", subcore_axis_name="s")
    run = pl.kernel(
        _sc_img_body,
        out_type=jax.ShapeDtypeStruct((S_img, B, h), jnp.float32),
        mesh=mesh,
        scratch_types=[
            pltpu.VMEM((2, h), jnp.float32),
            [pltpu.VMEM((1, SC_NB, h), jnp.float32) for _ in range(NBUF)],
            [pltpu.SemaphoreType.DMA for _ in range(NBUF)],
            [pltpu.SemaphoreType.DMA for _ in range(NBUF)],
        ],
    )
    out_t = run(img_t, mod_embed_table)
    return jnp.transpose(out_t, (1, 0, 2))


def kernel(poi_embedding, img_embedding, mod_embed_table):
    img_out = _sc_img_only(img_embedding, mod_embed_table)
    poi_out = _tc_poi_only(poi_embedding, mod_embed_table)
    return poi_out, img_out


# speedup vs baseline: 1.1110x; 1.0007x over previous
import jax
import jax.numpy as jnp
from jax import lax
import jax.experimental.pallas as pl
from jax.experimental.pallas import tpu as pltpu
from jax.experimental.pallas import tpu_sc as plsc

H = 128
BATCH_BLK = 128

_SC_INFO = plsc.get_sparse_core_info()
NC = _SC_INFO.num_cores
NS = _SC_INFO.num_subcores
NW = NC * NS

NBUF = 4
SC_NB = 32
SC_ROWS = 5


def _tc_poi_kernel(poi_ref, tbl_ref, poi_out_ref):
    poi_out_ref[...] = poi_ref[...] + tbl_ref[0:1, :][None]


def _tc_poi_only(poi_embedding, mod_embed_table):
    B, S_poi, h = poi_embedding.shape
    return pl.pallas_call(
        _tc_poi_kernel,
        grid=(B // BATCH_BLK,),
        in_specs=[
            pl.BlockSpec((BATCH_BLK, S_poi, h), lambda i: (i, 0, 0)),
            pl.BlockSpec((2, h), lambda i: (0, 0)),
        ],
        out_specs=pl.BlockSpec((BATCH_BLK, S_poi, h), lambda i: (i, 0, 0)),
        out_shape=jax.ShapeDtypeStruct(poi_embedding.shape, poi_embedding.dtype),
    )(poi_embedding, mod_embed_table)


def _sc_img_body(img_hbm, tbl_hbm, img_out, tbl_v, bufs, sin, sout):
    wid = lax.axis_index("s") * NC + lax.axis_index("c")
    S, B, _ = img_hbm.shape
    base = wid * SC_NB
    n = S // SC_ROWS

    pltpu.sync_copy(tbl_hbm, tbl_v)
    row1 = [tbl_v[1, pl.ds(h * 16, 16)] for h in range(H // 16)]

    def start_in(c):
        pltpu.async_copy(
            img_hbm.at[pl.ds(c * SC_ROWS, SC_ROWS), pl.ds(base, SC_NB)],
            bufs[c % NBUF],
            sin[c % NBUF],
        )

    def wait_in(c):
        pltpu.make_async_copy(
            img_hbm.at[pl.ds(c * SC_ROWS, SC_ROWS), pl.ds(base, SC_NB)],
            bufs[c % NBUF],
            sin[c % NBUF],
        ).wait()

    def start_out(c):
        pltpu.async_copy(
            bufs[c % NBUF],
            img_out.at[pl.ds(c * SC_ROWS, SC_ROWS), pl.ds(base, SC_NB)],
            sout[c % NBUF],
        )

    def wait_out(c):
        pltpu.make_async_copy(
            bufs[c % NBUF],
            img_out.at[pl.ds(c * SC_ROWS, SC_ROWS), pl.ds(base, SC_NB)],
            sout[c % NBUF],
        ).wait()

    def compute(c):
        buf = bufs[c % NBUF]

        @plsc.parallel_loop(0, SC_NB, unroll=4)
        def _batch(b):
            for r in range(SC_ROWS):
                for h in range(H // 16):
                    sl = pl.ds(h * 16, 16)
                    buf[r, b, sl] = buf[r, b, sl] + row1[h]

    for k in range(min(NBUF - 1, n)):
        start_in(k)
    for c in range(n):
        wait_in(c)
        nxt = c + NBUF - 1
        if nxt < n:
            if c >= 1:
                wait_out(c - 1)
            start_in(nxt)
        compute(c)
        start_out(c)
    for c in range(max(0, n - NBUF + 1), n):
        wait_out(c)


def _sc_img_only(img_embedding, mod_embed_table):
    B, S_img, h = img_embedding.shape
    img_t = jnp.transpose(img_embedding, (1, 0, 2))
    mesh = plsc.VectorSubcoreMesh(core_axis_name="c", subcore_axis_name="s")
    run = pl.kernel(
        _sc_img_body,
        out_type=jax.ShapeDtypeStruct((S_img, B, h), jnp.float32),
        mesh=mesh,
        scratch_types=[
            pltpu.VMEM((2, h), jnp.float32),
            [pltpu.VMEM((SC_ROWS, SC_NB, h), jnp.float32) for _ in range(NBUF)],
            [pltpu.SemaphoreType.DMA for _ in range(NBUF)],
            [pltpu.SemaphoreType.DMA for _ in range(NBUF)],
        ],
    )
    out_t = run(img_t, mod_embed_table)
    return jnp.transpose(out_t, (1, 0, 2))


def kernel(poi_embedding, img_embedding, mod_embed_table):
    img_out = _sc_img_only(img_embedding, mod_embed_table)
    poi_out = _tc_poi_only(poi_embedding, mod_embed_table)
    return poi_out, img_out
